# baseline (speedup 1.0000x reference)
import functools

import jax
import jax.numpy as jnp
from jax.experimental import pallas as pl
from jax.experimental.pallas import tpu as pltpu

_SLOPE = 0.2


def _pick_row_block(n, cap):
    best = 0
    for d in range(8, cap + 1, 8):
        if n % d == 0:
            best = d
    return best if best else n


def _proj_kernel(elu_in, x_ref, w_ref, asrc_ref, adst_ref,
                 h_ref, f1_ref, f2t_ref, hsum_ref):
    xb = x_ref[...]
    if elu_in:
        xb = jnp.where(xb > 0, xb, jnp.exp(xb) - 1.0)
    hb = jnp.dot(xb, w_ref[...], preferred_element_type=jnp.float32)
    h_ref[...] = hb
    f1_ref[...] = jnp.dot(hb, asrc_ref[...], preferred_element_type=jnp.float32)
    f2t_ref[0] = jax.lax.dot_general(
        adst_ref[...], hb,
        dimension_numbers=(((0,), (1,)), ((), ())),
        preferred_element_type=jnp.float32,
    )
    colsum = jnp.sum(hb, axis=0, keepdims=True)

    @pl.when(pl.program_id(0) == 0)
    def _():
        hsum_ref[...] = colsum

    @pl.when(pl.program_id(0) != 0)
    def _():
        hsum_ref[...] += colsum


def _proj(x, W, a_src, a_dst, elu_in):
    n, d_in = x.shape
    d_out = W.shape[1]
    br = _pick_row_block(n, 1000)
    nb = n // br
    return pl.pallas_call(
        functools.partial(_proj_kernel, elu_in),
        grid=(nb,),
        in_specs=[
            pl.BlockSpec((br, d_in), lambda i: (i, 0)),
            pl.BlockSpec((d_in, d_out), lambda i: (0, 0)),
            pl.BlockSpec((d_out, 1), lambda i: (0, 0)),
            pl.BlockSpec((d_out, 1), lambda i: (0, 0)),
        ],
        out_specs=[
            pl.BlockSpec((br, d_out), lambda i: (i, 0)),
            pl.BlockSpec((br, 1), lambda i: (i, 0)),
            pl.BlockSpec((1, 1, br), lambda i: (i, 0, 0)),
            pl.BlockSpec((1, d_out), lambda i: (0, 0)),
        ],
        out_shape=[
            jax.ShapeDtypeStruct((n, d_out), jnp.float32),
            jax.ShapeDtypeStruct((n, 1), jnp.float32),
            jax.ShapeDtypeStruct((nb, 1, br), jnp.float32),
            jax.ShapeDtypeStruct((1, d_out), jnp.float32),
        ],
    )(x, W, a_src, a_dst)


def _attn_kernel(n, nj, bc, adj_ref, h_ref, f1_ref, f2t_ref, hsum_ref, o_ref,
                 acc, l_s, f1s_s, f1t_s, f2b_s):
    j = pl.program_id(1)

    @pl.when(j == 0)
    def _():
        f2max = jnp.max(f2t_ref[...])
        w = f1_ref[...] + f2max
        m = jnp.where(w >= 0, w, _SLOPE * w)
        f1s_s[...] = f1_ref[...] - m
        f1t_s[...] = _SLOPE * f1_ref[...] - m
        f2b_s[...] = _SLOPE * f2t_ref[...]
        l_s[...] = jnp.zeros_like(l_s)
        acc[...] = jnp.zeros_like(acc)

    u = f1s_s[...] + f2t_ref[:, pl.ds(j * bc, bc)]
    t = f1t_s[...] + f2b_s[:, pl.ds(j * bc, bc)]
    p0 = jnp.exp2(jnp.maximum(u, t))
    hj = h_ref[pl.ds(j * bc, bc), :]

    @pl.when(j < nj - 1)
    def _():
        p = p0 * adj_ref[...]
        l_s[...] += jnp.sum(p, axis=1, keepdims=True)
        acc[...] += jnp.dot(p, hj, preferred_element_type=jnp.float32)

    @pl.when(j == nj - 1)
    def _():
        p = jnp.where(adj_ref[...] > 0, p0, 0.0)
        l = l_s[...] + jnp.sum(p, axis=1, keepdims=True)
        a = acc[...] + jnp.dot(p, hj, preferred_element_type=jnp.float32)
        o_ref[...] = jnp.where(l > 0, a / l, hsum_ref[...] * (1.0 / n))


def _attn(adj, h, f1, f2t, hsum):
    n, d = h.shape
    br = _pick_row_block(n, 512)
    bc = 2048
    ni = n // br
    nj = pl.cdiv(n, bc)
    n_pad = nj * bc
    h_p = jnp.pad(h, ((0, n_pad - n), (0, 0)))
    f2t_p = jnp.pad(f2t, ((0, 0), (0, n_pad - n)), constant_values=-1e30)
    return pl.pallas_call(
        functools.partial(_attn_kernel, n, nj, bc),
        grid=(ni, nj),
        in_specs=[
            pl.BlockSpec((br, bc), lambda i, j: (i, j)),
            pl.BlockSpec((n_pad, d), lambda i, j: (0, 0)),
            pl.BlockSpec((br, 1), lambda i, j: (i, 0)),
            pl.BlockSpec((1, n_pad), lambda i, j: (0, 0)),
            pl.BlockSpec((1, d), lambda i, j: (0, 0)),
        ],
        out_specs=pl.BlockSpec((br, d), lambda i, j: (i, 0)),
        out_shape=jax.ShapeDtypeStruct((n, d), jnp.float32),
        scratch_shapes=[
            pltpu.VMEM((br, d), jnp.float32),
            pltpu.VMEM((br, 1), jnp.float32),
            pltpu.VMEM((br, 1), jnp.float32),
            pltpu.VMEM((br, 1), jnp.float32),
            pltpu.VMEM((1, n_pad), jnp.float32),
        ],
        compiler_params=pltpu.CompilerParams(
            dimension_semantics=("arbitrary", "arbitrary"),
        ),
    )(adj, h_p, f1, f2t_p, hsum)


_LOG2E = 1.4426950408889634


def _gat_layer(adj, x, W, a, elu_in):
    d_out = W.shape[2]
    a_src = a[0, :d_out, :] * _LOG2E
    a_dst = a[0, d_out:, :] * _LOG2E
    h, f1, f2t, hsum = _proj(x, W[0], a_src, a_dst, elu_in)
    nb = f2t.shape[0]
    return _attn(adj, h, f1, f2t.reshape(1, nb * f2t.shape[2]), hsum)


def kernel(adj, x, W1, a1, W2, a2):
    h1 = _gat_layer(adj, x, W1, a1, elu_in=False)
    return _gat_layer(adj, h1, W2, a2, elu_in=True)


# speedup vs baseline: 1.0813x; 1.0813x over previous
import functools

import jax
import jax.numpy as jnp
from jax.experimental import pallas as pl
from jax.experimental.pallas import tpu as pltpu

_SLOPE = 0.2


def _pick_row_block(n, cap):
    best = 0
    for d in range(8, cap + 1, 8):
        if n % d == 0:
            best = d
    return best if best else n


def _proj_kernel(elu_in, x_ref, w_ref, asrc_ref, adst_ref,
                 h_ref, f1_ref, f2t_ref, hsum_ref):
    xb = x_ref[...]
    if elu_in:
        xb = jnp.where(xb > 0, xb, jnp.exp(xb) - 1.0)
    hb = jnp.dot(xb, w_ref[...], preferred_element_type=jnp.float32)
    h_ref[...] = hb
    f1_ref[...] = jnp.dot(hb, asrc_ref[...], preferred_element_type=jnp.float32)
    f2t_ref[0] = jax.lax.dot_general(
        adst_ref[...], hb,
        dimension_numbers=(((0,), (1,)), ((), ())),
        preferred_element_type=jnp.float32,
    )
    colsum = jnp.sum(hb, axis=0, keepdims=True)

    @pl.when(pl.program_id(0) == 0)
    def _():
        hsum_ref[...] = colsum

    @pl.when(pl.program_id(0) != 0)
    def _():
        hsum_ref[...] += colsum


def _proj(x, W, a_src, a_dst, elu_in):
    n, d_in = x.shape
    d_out = W.shape[1]
    br = _pick_row_block(n, 1000)
    nb = n // br
    return pl.pallas_call(
        functools.partial(_proj_kernel, elu_in),
        grid=(nb,),
        in_specs=[
            pl.BlockSpec((br, d_in), lambda i: (i, 0)),
            pl.BlockSpec((d_in, d_out), lambda i: (0, 0)),
            pl.BlockSpec((d_out, 1), lambda i: (0, 0)),
            pl.BlockSpec((d_out, 1), lambda i: (0, 0)),
        ],
        out_specs=[
            pl.BlockSpec((br, d_out), lambda i: (i, 0)),
            pl.BlockSpec((br, 1), lambda i: (i, 0)),
            pl.BlockSpec((1, 1, br), lambda i: (i, 0, 0)),
            pl.BlockSpec((1, d_out), lambda i: (0, 0)),
        ],
        out_shape=[
            jax.ShapeDtypeStruct((n, d_out), jnp.float32),
            jax.ShapeDtypeStruct((n, 1), jnp.float32),
            jax.ShapeDtypeStruct((nb, 1, br), jnp.float32),
            jax.ShapeDtypeStruct((1, d_out), jnp.float32),
        ],
    )(x, W, a_src, a_dst)


def _attn_kernel(n, nj, bc, adj_ref, h_ref, f1_ref, f2t_ref, hsum_ref, o_ref,
                 acc, l_s, f1s_s, f1t_s, f2b_s):
    j = pl.program_id(1)

    @pl.when(j == 0)
    def _():
        f2max = jnp.max(f2t_ref[...])
        w = f1_ref[...] + f2max
        m = jnp.where(w >= 0, w, _SLOPE * w)
        f1s_s[...] = f1_ref[...] - m
        f1t_s[...] = _SLOPE * f1_ref[...] - m
        f2b_s[...] = _SLOPE * f2t_ref[...]
        l_s[...] = jnp.zeros_like(l_s)
        acc[...] = jnp.zeros_like(acc)

    u = f1s_s[...] + f2t_ref[:, pl.ds(j * bc, bc)]
    t = f1t_s[...] + f2b_s[:, pl.ds(j * bc, bc)]
    p0 = jnp.exp2(jnp.maximum(u, t))
    hj = h_ref[pl.ds(j * bc, bc), :]
    p = jnp.where(adj_ref[...] > 0, p0, 0.0)
    l_s[...] += jnp.sum(p, axis=1, keepdims=True)
    acc[...] += jnp.dot(p, hj, preferred_element_type=jnp.float32)

    @pl.when(j == nj - 1)
    def _():
        l = l_s[...]
        o_ref[...] = jnp.where(l > 0, acc[...] / l, hsum_ref[...] * (1.0 / n))


def _attn(adj, h, f1, f2t, hsum):
    n, d = h.shape
    br = _pick_row_block(n, 512)
    bc = 2048
    ni = n // br
    nj = pl.cdiv(n, bc)
    n_pad = nj * bc
    h_p = jnp.pad(h, ((0, n_pad - n), (0, 0)))
    f2t_p = jnp.pad(f2t, ((0, 0), (0, n_pad - n)), constant_values=-1e30)
    return pl.pallas_call(
        functools.partial(_attn_kernel, n, nj, bc),
        grid=(ni, nj),
        in_specs=[
            pl.BlockSpec((br, bc), lambda i, j: (i, j)),
            pl.BlockSpec((n_pad, d), lambda i, j: (0, 0)),
            pl.BlockSpec((br, 1), lambda i, j: (i, 0)),
            pl.BlockSpec((1, n_pad), lambda i, j: (0, 0)),
            pl.BlockSpec((1, d), lambda i, j: (0, 0)),
        ],
        out_specs=pl.BlockSpec((br, d), lambda i, j: (i, 0)),
        out_shape=jax.ShapeDtypeStruct((n, d), jnp.float32),
        scratch_shapes=[
            pltpu.VMEM((br, d), jnp.float32),
            pltpu.VMEM((br, 1), jnp.float32),
            pltpu.VMEM((br, 1), jnp.float32),
            pltpu.VMEM((br, 1), jnp.float32),
            pltpu.VMEM((1, n_pad), jnp.float32),
        ],
        compiler_params=pltpu.CompilerParams(
            dimension_semantics=("arbitrary", "arbitrary"),
        ),
    )(adj, h_p, f1, f2t_p, hsum)


_LOG2E = 1.4426950408889634


def _gat_layer(adj, x, W, a, elu_in):
    d_out = W.shape[2]
    a_src = a[0, :d_out, :] * _LOG2E
    a_dst = a[0, d_out:, :] * _LOG2E
    h, f1, f2t, hsum = _proj(x, W[0], a_src, a_dst, elu_in)
    nb = f2t.shape[0]
    return _attn(adj, h, f1, f2t.reshape(1, nb * f2t.shape[2]), hsum)


def kernel(adj, x, W1, a1, W2, a2):
    h1 = _gat_layer(adj, x, W1, a1, elu_in=False)
    return _gat_layer(adj, h1, W2, a2, elu_in=True)
